# baseline (speedup 1.0000x reference)
import numpy as np
import jax
import jax.numpy as jnp
from jax import lax
from jax.experimental import pallas as pl
from jax.experimental.pallas import tpu as pltpu

_TOPN = 100
_B = 4
_K = 8
_BIGN = np.int32(2**30)
_NEG = float("-inf")

_ANCHORS = np.array(
    [[-38., -16., 0., 53., 31., 15.],
     [-84., -40., 0., 99., 55., 15.],
     [-176., -88., 0., 191., 103., 15.],
     [-360., -184., 0., 375., 199., 15.],
     [-24., -24., 0., 39., 39., 15.],
     [-56., -56., 0., 71., 71., 15.],
     [-120., -120., 0., 135., 135., 15.],
     [-248., -248., 0., 263., 263., 15.],
     [-14., -36., 0., 29., 51., 15.],
     [-36., -80., 0., 51., 95., 15.],
     [-80., -168., 0., 95., 183., 15.],
     [-168., -344., 0., 183., 359., 15.]],
    dtype=np.float32)


def _proposal_kernel(scores_ref, bbox_ref, im_ref, out_ref,
                     selv_ref, seln_ref, *scratches):
    riota = lax.broadcasted_iota(jnp.int32, (192, 512), 0)
    m3 = riota * 512 + lax.broadcasted_iota(jnp.int32, (192, 512), 1)
    a3 = m3 // 8192
    n3 = (m3 - a3 * 8192) * 12 + a3
    lane = lax.broadcasted_iota(jnp.int32, (1, 128), 1)

    Cs, Cns = [], []
    for b in range(_B):
        S = scores_ref[b]
        Tv, Tn = [], []
        for t in range(_K):
            cmax = jnp.max(S, axis=0, keepdims=True)
            hit = S == cmax
            minrow = jnp.min(jnp.where(hit, riota, _BIGN),
                             axis=0, keepdims=True)
            knock = hit & (riota == minrow)
            ncol = jnp.min(jnp.where(knock, n3, _BIGN),
                           axis=0, keepdims=True)
            Tv.append(cmax)
            Tn.append(ncol)
            if t < _K - 1:
                S = jnp.where(knock, _NEG, S)
        Cs.append(jnp.concatenate(Tv, axis=0))
        Cns.append(jnp.concatenate(Tn, axis=0))

    def body(i, carry):
        F, Fn, D, selv, seln = [list(x) for x in carry]
        for b in range(_B):
            v = jnp.max(F[b], keepdims=True).reshape(1, 1)
            nsel = jnp.min(jnp.where(F[b] == v, Fn[b], _BIGN),
                           keepdims=True).reshape(1, 1)
            selv[b] = jnp.where(lane == i, v, selv[b])
            seln[b] = jnp.where(lane == i, nsel, seln[b])
            hit = Fn[b] == nsel
            D[b] = D[b] + hit.astype(jnp.int32)
            newv = jnp.full((1, 512), _NEG, jnp.float32)
            newn = jnp.full((1, 512), _BIGN, jnp.int32)
            for t in range(1, _K):
                m = D[b] == t
                newv = jnp.where(m, Cs[b][t:t + 1, :], newv)
                newn = jnp.where(m, Cns[b][t:t + 1, :], newn)
            F[b] = jnp.where(hit, newv, F[b])
            Fn[b] = jnp.where(hit, newn, Fn[b])
        return tuple(F), tuple(Fn), tuple(D), tuple(selv), tuple(seln)

    selv0 = tuple(jnp.zeros((1, 128), jnp.float32) for _ in range(_B))
    seln0 = tuple(jnp.zeros((1, 128), jnp.int32) for _ in range(_B))
    F0 = tuple(Cs[b][0:1, :] for b in range(_B))
    Fn0 = tuple(Cns[b][0:1, :] for b in range(_B))
    D0 = tuple(jnp.zeros((1, 512), jnp.int32) for _ in range(_B))
    _, _, Df, selv, seln = lax.fori_loop(
        0, _TOPN, body, (F0, Fn0, D0, selv0, seln0))

    worst = jnp.zeros((1, 1), jnp.int32)
    for b in range(_B):
        worst = jnp.maximum(worst,
                            jnp.max(Df[b], keepdims=True).reshape(1, 1))
    for b in range(_B):
        selv_ref[pl.ds(b, 1), :] = selv[b]
        seln_ref[pl.ds(b, 1), :] = seln[b]

    @pl.when(worst[0, 0] >= _K)
    def _fallback():
        for b in range(_B):
            scratches[b][...] = scores_ref[b]

        def fbody(i, carry):
            fv, fn = [list(x) for x in carry]
            for b in range(_B):
                S = scratches[b][...]
                v = jnp.max(S, keepdims=True).reshape(1, 1)
                nsel = jnp.min(jnp.where(S == v, n3, _BIGN),
                               keepdims=True).reshape(1, 1)
                fv[b] = jnp.where(lane == i, v, fv[b])
                fn[b] = jnp.where(lane == i, nsel, fn[b])
                scratches[b][...] = jnp.where(n3 == nsel, _NEG, S)
            return tuple(fv), tuple(fn)

        fv, fn = lax.fori_loop(0, _TOPN, fbody, (selv0, seln0))
        for b in range(_B):
            selv_ref[pl.ds(b, 1), :] = fv[b]
            seln_ref[pl.ds(b, 1), :] = fn[b]

    for b in range(_B):
        seln_b = seln_ref[pl.ds(b, 1), :]
        selv_b = selv_ref[pl.ds(b, 1), :]
        p_i = seln_b // 12
        a_i = seln_b - p_i * 12
        G = jnp.zeros((72, 128), jnp.float32)
        for k in range(8):
            pio = lax.broadcasted_iota(jnp.int32, (1024, 128), 0) + k * 1024
            oneh = (pio == p_i).astype(jnp.float32)
            blk = bbox_ref[b, :, k * 1024:(k + 1) * 1024]
            G = G + lax.dot_general(blk, oneh, (((1,), (0,)), ((), ())),
                                    preferred_element_type=jnp.float32)
        d = jnp.zeros((6, 128), jnp.float32)
        an = [jnp.zeros((1, 128), jnp.float32) for _ in range(6)]
        for a in range(12):
            hit_a = a_i == a
            d = jnp.where(hit_a, G[a * 6:(a + 1) * 6, :], d)
            for jj in range(6):
                an[jj] = jnp.where(hit_a, float(_ANCHORS[a, jj]), an[jj])

        hs = p_i // 256
        ws = (p_i // 8) % 32
        ts = p_i % 8
        sx = (ws * 16).astype(jnp.float32)
        sy = (hs * 16).astype(jnp.float32)
        sz = ts.astype(jnp.float32)
        a0 = an[0] + sx
        a1 = an[1] + sy
        a2 = an[2] + sz
        a3_ = an[3] + sx
        a4 = an[4] + sy
        a5 = an[5] + sz
        w = a3_ - a0 + 1.0
        h = a4 - a1 + 1.0
        l = a5 - a2 + 1.0
        cx = a0 + 0.5 * w
        cy = a1 + 0.5 * h
        ct = a2 + 0.5 * l
        pcx = d[0:1, :] * w + cx
        pcy = d[1:2, :] * h + cy
        pct = d[2:3, :] * l + ct
        pw = jnp.exp(d[3:4, :]) * w
        ph = jnp.exp(d[4:5, :]) * h
        pll = jnp.exp(d[5:6, :]) * l
        Hc = im_ref[b, 0] - 1.0
        Wc = im_ref[b, 1] - 1.0
        Tc = im_ref[b, 2] - 1.0
        x1 = jnp.clip(pcx - 0.5 * pw, 0.0, Wc)
        y1 = jnp.clip(pcy - 0.5 * ph, 0.0, Hc)
        t1 = jnp.clip(pct - 0.5 * pll, 0.0, Tc)
        x2 = jnp.clip(pcx + 0.5 * pw, 0.0, Wc)
        y2 = jnp.clip(pcy + 0.5 * ph, 0.0, Hc)
        t2 = jnp.clip(pct + 0.5 * pll, 0.0, Tc)
        brow = jnp.full((1, 128), float(b), jnp.float32)
        out_ref[b] = jnp.concatenate(
            [brow, x1, y1, t1, x2, y2, t2, selv_b], axis=0)


def kernel(scores_full, bbox_frame, im_info):
    B = scores_full.shape[0]
    scores = scores_full[:, 12:, :, :, :].reshape(B, 192, 512)
    bbox = bbox_frame.reshape(B, 72, 8192)
    out = pl.pallas_call(
        _proposal_kernel,
        in_specs=[
            pl.BlockSpec((B, 192, 512), lambda: (0, 0, 0)),
            pl.BlockSpec((B, 72, 8192), lambda: (0, 0, 0)),
            pl.BlockSpec(memory_space=pltpu.SMEM),
        ],
        out_specs=pl.BlockSpec((B, 8, 128), lambda: (0, 0, 0)),
        out_shape=jax.ShapeDtypeStruct((B, 8, 128), jnp.float32),
        scratch_shapes=[pltpu.VMEM((8, 128), jnp.float32),
                        pltpu.VMEM((8, 128), jnp.int32)]
                       + [pltpu.VMEM((192, 512), jnp.float32)
                          for _ in range(B)],
    )(scores, bbox, im_info)
    return out[:, :, :_TOPN].transpose(0, 2, 1)


# speedup vs baseline: 2.5436x; 2.5436x over previous
import numpy as np
import jax
import jax.numpy as jnp
from jax import lax
from jax.experimental import pallas as pl
from jax.experimental.pallas import tpu as pltpu

_TOPN = 100
_B = 4
_K = 8
_BIGN = np.int32(2**30)
_NEG = float("-inf")

_ANCHORS = np.array(
    [[-38., -16., 0., 53., 31., 15.],
     [-84., -40., 0., 99., 55., 15.],
     [-176., -88., 0., 191., 103., 15.],
     [-360., -184., 0., 375., 199., 15.],
     [-24., -24., 0., 39., 39., 15.],
     [-56., -56., 0., 71., 71., 15.],
     [-120., -120., 0., 135., 135., 15.],
     [-248., -248., 0., 263., 263., 15.],
     [-14., -36., 0., 29., 51., 15.],
     [-36., -80., 0., 51., 95., 15.],
     [-80., -168., 0., 95., 183., 15.],
     [-168., -344., 0., 183., 359., 15.]],
    dtype=np.float32)


def _proposal_kernel(scores_ref, bbox_ref, im_ref, out_ref,
                     selv_ref, seln_ref, *scratches):
    riota = lax.broadcasted_iota(jnp.int32, (192, 512), 0)
    m3 = riota * 512 + lax.broadcasted_iota(jnp.int32, (192, 512), 1)
    a3 = m3 // 8192
    n3 = (m3 - a3 * 8192) * 12 + a3
    lane = lax.broadcasted_iota(jnp.int32, (1, 128), 1)

    Cs, Cns = [], []
    for b in range(_B):
        S = scores_ref[b]
        Tv, Tn = [], []
        for t in range(_K):
            cmax = jnp.max(S, axis=0, keepdims=True)
            hit = S == cmax
            ncol = jnp.min(jnp.where(hit, n3, _BIGN),
                           axis=0, keepdims=True)
            knock = hit & (n3 == ncol)
            Tv.append(cmax)
            Tn.append(ncol)
            if t < _K - 1:
                S = jnp.where(knock, _NEG, S)
        Cs.append(jnp.concatenate(Tv, axis=0))
        Cns.append(jnp.concatenate(Tn, axis=0))

    def cmp_merge(Sv, Sn, k, dm):
        n2, W = Sv.shape
        r = n2 // (2 * k)
        v4 = Sv.reshape(r, 2, k, W)
        n4 = Sn.reshape(r, 2, k, W)
        av, bv = v4[:, 0], v4[:, 1]
        an, bn = n4[:, 0], n4[:, 1]
        bet = (av > bv) | ((av == bv) & (an < bn))
        bet = bet != (~dm)
        hv = jnp.where(bet, av, bv)
        hn = jnp.where(bet, an, bn)
        lv = jnp.where(bet, bv, av)
        ln_ = jnp.where(bet, bn, an)
        Sv = jnp.concatenate([hv[:, None], lv[:, None]], axis=1)
        Sn = jnp.concatenate([hn[:, None], ln_[:, None]], axis=1)
        return Sv.reshape(n2, W), Sn.reshape(n2, W)

    worst = jnp.zeros((1, 1), jnp.int32)
    for b in range(_B):
        Vv, Vn = Cs[b], Cns[b]
        Vv = jnp.concatenate(
            [Vv[:, :256]] + [Vv[t:t + 1, 256:] for t in range(7, -1, -1)],
            axis=0).reshape(16, 256)
        Vn = jnp.concatenate(
            [Vn[:, :256]] + [Vn[t:t + 1, 256:] for t in range(7, -1, -1)],
            axis=0).reshape(16, 256)
        W = 256
        while W >= 1:
            L2 = Vv.shape[0]
            dm = (lax.broadcasted_iota(jnp.int32, (1, 1, W), 2)
                  < max(W // 2, 1))
            k = L2 // 2
            while k >= 1:
                Vv, Vn = cmp_merge(Vv, Vn, k, dm)
                k //= 2
            if Vv.shape[0] > 128:
                keep = dm.reshape(1, W)
                Vv = jnp.where(keep, Vv[:128], Vv[128:])
                Vn = jnp.where(keep, Vn[:128], Vn[128:])
            if W == 1:
                break
            W2 = W // 2
            Vv = jnp.concatenate([Vv[:, :W2], Vv[:, W2:]], axis=0)
            Vn = jnp.concatenate([Vn[:, :W2], Vn[:, W2:]], axis=0)
            W = W2
        selv_b = Vv.reshape(1, 128)
        seln_b = Vn.reshape(1, 128)
        selv_ref[pl.ds(b, 1), :] = selv_b
        seln_ref[pl.ds(b, 1), :] = seln_b
        v100 = jnp.max(jnp.where(lane == _TOPN - 1, selv_b, _NEG),
                       keepdims=True).reshape(1, 1)
        n100 = jnp.min(jnp.where(lane == _TOPN - 1, seln_b, _BIGN),
                       keepdims=True).reshape(1, 1)
        c8v, c8n = Cs[b][_K - 1:_K, :], Cns[b][_K - 1:_K, :]
        bad = (c8v > v100) | ((c8v == v100) & (c8n < n100))
        worst = jnp.maximum(
            worst, jnp.max(bad.astype(jnp.int32), keepdims=True)
            .reshape(1, 1))

    selv0 = tuple(jnp.zeros((1, 128), jnp.float32) for _ in range(_B))
    seln0 = tuple(jnp.zeros((1, 128), jnp.int32) for _ in range(_B))

    @pl.when(worst[0, 0] >= 1)
    def _fallback():
        for b in range(_B):
            scratches[b][...] = scores_ref[b]

        def fbody(i, carry):
            fv, fn = [list(x) for x in carry]
            for b in range(_B):
                S = scratches[b][...]
                v = jnp.max(S, keepdims=True).reshape(1, 1)
                nsel = jnp.min(jnp.where(S == v, n3, _BIGN),
                               keepdims=True).reshape(1, 1)
                fv[b] = jnp.where(lane == i, v, fv[b])
                fn[b] = jnp.where(lane == i, nsel, fn[b])
                scratches[b][...] = jnp.where(n3 == nsel, _NEG, S)
            return tuple(fv), tuple(fn)

        fv, fn = lax.fori_loop(0, _TOPN, fbody, (selv0, seln0))
        for b in range(_B):
            selv_ref[pl.ds(b, 1), :] = fv[b]
            seln_ref[pl.ds(b, 1), :] = fn[b]

    for b in range(_B):
        seln_b = seln_ref[pl.ds(b, 1), :]
        selv_b = selv_ref[pl.ds(b, 1), :]
        p_i = seln_b // 12
        a_i = seln_b - p_i * 12
        G = jnp.zeros((72, 128), jnp.float32)
        for k in range(8):
            pio = lax.broadcasted_iota(jnp.int32, (1024, 128), 0) + k * 1024
            oneh = (pio == p_i).astype(jnp.float32)
            blk = bbox_ref[b, :, k * 1024:(k + 1) * 1024]
            G = G + lax.dot_general(blk, oneh, (((1,), (0,)), ((), ())),
                                    preferred_element_type=jnp.float32)
        d = jnp.zeros((6, 128), jnp.float32)
        an = [jnp.zeros((1, 128), jnp.float32) for _ in range(6)]
        for a in range(12):
            hit_a = a_i == a
            d = jnp.where(hit_a, G[a * 6:(a + 1) * 6, :], d)
            for jj in range(6):
                an[jj] = jnp.where(hit_a, float(_ANCHORS[a, jj]), an[jj])

        hs = p_i // 256
        ws = (p_i // 8) % 32
        ts = p_i % 8
        sx = (ws * 16).astype(jnp.float32)
        sy = (hs * 16).astype(jnp.float32)
        sz = ts.astype(jnp.float32)
        a0 = an[0] + sx
        a1 = an[1] + sy
        a2 = an[2] + sz
        a3_ = an[3] + sx
        a4 = an[4] + sy
        a5 = an[5] + sz
        w = a3_ - a0 + 1.0
        h = a4 - a1 + 1.0
        l = a5 - a2 + 1.0
        cx = a0 + 0.5 * w
        cy = a1 + 0.5 * h
        ct = a2 + 0.5 * l
        pcx = d[0:1, :] * w + cx
        pcy = d[1:2, :] * h + cy
        pct = d[2:3, :] * l + ct
        pw = jnp.exp(d[3:4, :]) * w
        ph = jnp.exp(d[4:5, :]) * h
        pll = jnp.exp(d[5:6, :]) * l
        Hc = im_ref[b, 0] - 1.0
        Wc = im_ref[b, 1] - 1.0
        Tc = im_ref[b, 2] - 1.0
        x1 = jnp.clip(pcx - 0.5 * pw, 0.0, Wc)
        y1 = jnp.clip(pcy - 0.5 * ph, 0.0, Hc)
        t1 = jnp.clip(pct - 0.5 * pll, 0.0, Tc)
        x2 = jnp.clip(pcx + 0.5 * pw, 0.0, Wc)
        y2 = jnp.clip(pcy + 0.5 * ph, 0.0, Hc)
        t2 = jnp.clip(pct + 0.5 * pll, 0.0, Tc)
        brow = jnp.full((1, 128), float(b), jnp.float32)
        out_ref[b] = jnp.concatenate(
            [brow, x1, y1, t1, x2, y2, t2, selv_b], axis=0)


def kernel(scores_full, bbox_frame, im_info):
    B = scores_full.shape[0]
    scores = scores_full[:, 12:, :, :, :].reshape(B, 192, 512)
    bbox = bbox_frame.reshape(B, 72, 8192)
    out = pl.pallas_call(
        _proposal_kernel,
        in_specs=[
            pl.BlockSpec((B, 192, 512), lambda: (0, 0, 0)),
            pl.BlockSpec((B, 72, 8192), lambda: (0, 0, 0)),
            pl.BlockSpec(memory_space=pltpu.SMEM),
        ],
        out_specs=pl.BlockSpec((B, 8, 128), lambda: (0, 0, 0)),
        out_shape=jax.ShapeDtypeStruct((B, 8, 128), jnp.float32),
        scratch_shapes=[pltpu.VMEM((8, 128), jnp.float32),
                        pltpu.VMEM((8, 128), jnp.int32)]
                       + [pltpu.VMEM((192, 512), jnp.float32)
                          for _ in range(B)],
    )(scores, bbox, im_info)
    return out[:, :, :_TOPN].transpose(0, 2, 1)


# speedup vs baseline: 2.9346x; 1.1537x over previous
import numpy as np
import jax
import jax.numpy as jnp
from jax import lax
from jax.experimental import pallas as pl
from jax.experimental.pallas import tpu as pltpu

_TOPN = 100
_B = 4
_K = 8
_BIGN = np.int32(2**30)
_NEG = float("-inf")

_ANCHORS = np.array(
    [[-38., -16., 0., 53., 31., 15.],
     [-84., -40., 0., 99., 55., 15.],
     [-176., -88., 0., 191., 103., 15.],
     [-360., -184., 0., 375., 199., 15.],
     [-24., -24., 0., 39., 39., 15.],
     [-56., -56., 0., 71., 71., 15.],
     [-120., -120., 0., 135., 135., 15.],
     [-248., -248., 0., 263., 263., 15.],
     [-14., -36., 0., 29., 51., 15.],
     [-36., -80., 0., 51., 95., 15.],
     [-80., -168., 0., 95., 183., 15.],
     [-168., -344., 0., 183., 359., 15.]],
    dtype=np.float32)


def _proposal_kernel(scores_ref, bbox_ref, im_ref, out_ref,
                     selv_ref, seln_ref, s_scratch):
    riota = lax.broadcasted_iota(jnp.int32, (192, 2048), 0)
    jiota = lax.broadcasted_iota(jnp.int32, (192, 2048), 1)
    c3 = jiota // 4
    m3 = riota * 512 + c3
    a3 = m3 // 8192
    n3 = (m3 - a3 * 8192) * 12 + a3
    lane = lax.broadcasted_iota(jnp.int32, (1, 128), 1)

    S = scores_ref[...]
    Tv, Tn = [], []
    for t in range(_K):
        cmax = jnp.max(S, axis=0, keepdims=True)
        hit = S == cmax
        ncol = jnp.min(jnp.where(hit, n3, _BIGN), axis=0, keepdims=True)
        Tv.append(cmax)
        Tn.append(ncol)
        if t < _K - 1:
            S = jnp.where(hit & (n3 == ncol), _NEG, S)
    Cv = jnp.concatenate(Tv, axis=0)
    Cn = jnp.concatenate(Tn, axis=0)

    def cmp_merge(Sv, Sn, k, dm):
        n2, W = Sv.shape
        r = n2 // (2 * k)
        v4 = Sv.reshape(r, 2, k, W)
        n4 = Sn.reshape(r, 2, k, W)
        av, bv = v4[:, 0], v4[:, 1]
        an, bn = n4[:, 0], n4[:, 1]
        bet = (av > bv) | ((av == bv) & (an < bn))
        bet = bet != (~dm)
        hv = jnp.where(bet, av, bv)
        hn = jnp.where(bet, an, bn)
        lv = jnp.where(bet, bv, av)
        ln_ = jnp.where(bet, bn, an)
        Sv = jnp.concatenate([hv[:, None], lv[:, None]], axis=1)
        Sn = jnp.concatenate([hn[:, None], ln_[:, None]], axis=1)
        return Sv.reshape(n2, W), Sn.reshape(n2, W)

    Vv = jnp.concatenate(
        [Cv[:, :1024]] + [Cv[t:t + 1, 1024:] for t in range(7, -1, -1)],
        axis=0)
    Vn = jnp.concatenate(
        [Cn[:, :1024]] + [Cn[t:t + 1, 1024:] for t in range(7, -1, -1)],
        axis=0)
    W = 1024
    while True:
        wb = W // 4
        dm = (lax.broadcasted_iota(jnp.int32, (1, 1, W), 2) // 4
              < max(wb // 2, 1))
        k = Vv.shape[0] // 2
        while k >= 1:
            Vv, Vn = cmp_merge(Vv, Vn, k, dm)
            k //= 2
        if Vv.shape[0] > 128:
            keep = dm.reshape(1, W)
            Vv = jnp.where(keep, Vv[:128], Vv[128:])
            Vn = jnp.where(keep, Vn[:128], Vn[128:])
        if W == 4:
            break
        W2 = W // 2
        Vv = jnp.concatenate([Vv[:, :W2], Vv[:, W2:]], axis=0)
        Vn = jnp.concatenate([Vn[:, :W2], Vn[:, W2:]], axis=0)
        W = W2

    for b in range(_B):
        selv_ref[pl.ds(b, 1), :] = Vv[:, b:b + 1].reshape(1, 128)
        seln_ref[pl.ds(b, 1), :] = Vn[:, b:b + 1].reshape(1, 128)

    v100 = Vv[_TOPN - 1:_TOPN, :]
    n100 = Vn[_TOPN - 1:_TOPN, :]
    for _ in range(9):
        v100 = jnp.concatenate([v100, v100], axis=1)
        n100 = jnp.concatenate([n100, n100], axis=1)
    c8v, c8n = Cv[_K - 1:_K, :], Cn[_K - 1:_K, :]
    bad = (c8v > v100) | ((c8v == v100) & (c8n < n100))
    worst = jnp.max(bad.astype(jnp.int32), keepdims=True).reshape(1, 1)

    selv0 = tuple(jnp.zeros((1, 128), jnp.float32) for _ in range(_B))
    seln0 = tuple(jnp.zeros((1, 128), jnp.int32) for _ in range(_B))

    @pl.when(worst[0, 0] >= 1)
    def _fallback():
        s_scratch[...] = scores_ref[...]
        bmasks = [(jiota % 4) == b for b in range(_B)]

        def fbody(i, carry):
            fv, fn = [list(x) for x in carry]
            Sf = s_scratch[...]
            for b in range(_B):
                v = jnp.max(jnp.where(bmasks[b], Sf, _NEG),
                            keepdims=True).reshape(1, 1)
                nsel = jnp.min(jnp.where(bmasks[b] & (Sf == v), n3, _BIGN),
                               keepdims=True).reshape(1, 1)
                fv[b] = jnp.where(lane == i, v, fv[b])
                fn[b] = jnp.where(lane == i, nsel, fn[b])
                Sf = jnp.where(bmasks[b] & (n3 == nsel), _NEG, Sf)
            s_scratch[...] = Sf
            return tuple(fv), tuple(fn)

        fv, fn = lax.fori_loop(0, _TOPN, fbody, (selv0, seln0))
        for b in range(_B):
            selv_ref[pl.ds(b, 1), :] = fv[b]
            seln_ref[pl.ds(b, 1), :] = fn[b]

    for b in range(_B):
        seln_b = seln_ref[pl.ds(b, 1), :]
        selv_b = selv_ref[pl.ds(b, 1), :]
        p_i = seln_b // 12
        a_i = seln_b - p_i * 12
        G = jnp.zeros((72, 128), jnp.float32)
        for k in range(8):
            pio = lax.broadcasted_iota(jnp.int32, (1024, 128), 0) + k * 1024
            oneh = (pio == p_i).astype(jnp.float32)
            blk = bbox_ref[b, :, k * 1024:(k + 1) * 1024]
            G = G + lax.dot_general(blk, oneh, (((1,), (0,)), ((), ())),
                                    preferred_element_type=jnp.float32)
        d = jnp.zeros((6, 128), jnp.float32)
        an = [jnp.zeros((1, 128), jnp.float32) for _ in range(6)]
        for a in range(12):
            hit_a = a_i == a
            d = jnp.where(hit_a, G[a * 6:(a + 1) * 6, :], d)
            for jj in range(6):
                an[jj] = jnp.where(hit_a, float(_ANCHORS[a, jj]), an[jj])

        hs = p_i // 256
        ws = (p_i // 8) % 32
        ts = p_i % 8
        sx = (ws * 16).astype(jnp.float32)
        sy = (hs * 16).astype(jnp.float32)
        sz = ts.astype(jnp.float32)
        a0 = an[0] + sx
        a1 = an[1] + sy
        a2 = an[2] + sz
        a3_ = an[3] + sx
        a4 = an[4] + sy
        a5 = an[5] + sz
        w = a3_ - a0 + 1.0
        h = a4 - a1 + 1.0
        l = a5 - a2 + 1.0
        cx = a0 + 0.5 * w
        cy = a1 + 0.5 * h
        ct = a2 + 0.5 * l
        pcx = d[0:1, :] * w + cx
        pcy = d[1:2, :] * h + cy
        pct = d[2:3, :] * l + ct
        pw = jnp.exp(d[3:4, :]) * w
        ph = jnp.exp(d[4:5, :]) * h
        pll = jnp.exp(d[5:6, :]) * l
        Hc = im_ref[b, 0] - 1.0
        Wc = im_ref[b, 1] - 1.0
        Tc = im_ref[b, 2] - 1.0
        x1 = jnp.clip(pcx - 0.5 * pw, 0.0, Wc)
        y1 = jnp.clip(pcy - 0.5 * ph, 0.0, Hc)
        t1 = jnp.clip(pct - 0.5 * pll, 0.0, Tc)
        x2 = jnp.clip(pcx + 0.5 * pw, 0.0, Wc)
        y2 = jnp.clip(pcy + 0.5 * ph, 0.0, Hc)
        t2 = jnp.clip(pct + 0.5 * pll, 0.0, Tc)
        brow = jnp.full((1, 128), float(b), jnp.float32)
        out_ref[b] = jnp.concatenate(
            [brow, x1, y1, t1, x2, y2, t2, selv_b], axis=0)


def kernel(scores_full, bbox_frame, im_info):
    B = scores_full.shape[0]
    scores = (scores_full[:, 12:, :, :, :].reshape(B, 192, 512)
              .transpose(1, 2, 0).reshape(192, 512 * B))
    bbox = bbox_frame.reshape(B, 72, 8192)
    out = pl.pallas_call(
        _proposal_kernel,
        in_specs=[
            pl.BlockSpec((192, 512 * B), lambda: (0, 0)),
            pl.BlockSpec((B, 72, 8192), lambda: (0, 0, 0)),
            pl.BlockSpec(memory_space=pltpu.SMEM),
        ],
        out_specs=pl.BlockSpec((B, 8, 128), lambda: (0, 0, 0)),
        out_shape=jax.ShapeDtypeStruct((B, 8, 128), jnp.float32),
        scratch_shapes=[pltpu.VMEM((8, 128), jnp.float32),
                        pltpu.VMEM((8, 128), jnp.int32),
                        pltpu.VMEM((192, 2048), jnp.float32)],
    )(scores, bbox, im_info)
    return out[:, :, :_TOPN].transpose(0, 2, 1)
